# baseline (speedup 1.0000x reference)
import jax
import jax.numpy as jnp
from jax import lax
from jax.experimental import pallas as pl
from jax.experimental.pallas import tpu as pltpu
from jax.experimental.pallas import tpu_sc as plsc

N = 10000
M = 32
D = 128
K = 16
NW = 32
C = 320
NPAD = NW * C
SPC = 4
EPS = SPC * M
NSTEP = C // SPC
NB = 2
MD = 4
NEDGE = N * M


def _sc_gather_kernel(atom_hbm, idx_hbm, bwi_hbm, bwj_hbm,
                      g_hbm,
                      table_sp,
                      ix0, ix1, ix2, ix3,
                      bi0, bi1, bi2, bi3,
                      bj0, bj1, bj2, bj3,
                      rows0, rows1, go0, go1,
                      gsem, osem, msem):
    ix = [ix0, ix1, ix2, ix3]
    bi = [bi0, bi1, bi2, bi3]
    bj = [bj0, bj1, bj2, bj3]
    rows = [rows0, rows1]
    go = [go0, go1]

    nc = 2
    wid = lax.axis_index("s") * nc + lax.axis_index("c")
    ebase = wid * (C * M)
    cbase = wid * C

    @pl.when(lax.axis_index("s") == 0)
    def _stage_table():
        pltpu.sync_copy(atom_hbm, table_sp)

    def meta_issue(g, p):
        e0 = ebase + g * EPS
        pltpu.async_copy(idx_hbm.at[pl.ds(e0, EPS)], ix[p], msem)
        pltpu.async_copy(bwi_hbm.at[pl.ds(e0, EPS)], bi[p], msem)
        pltpu.async_copy(bwj_hbm.at[pl.ds(e0, EPS)], bj[p], msem)

    def meta_wait(p):
        pltpu.make_async_copy(idx_hbm.at[pl.ds(0, EPS)], ix[p], msem).wait()
        pltpu.make_async_copy(bwi_hbm.at[pl.ds(0, EPS)], bi[p], msem).wait()
        pltpu.make_async_copy(bwj_hbm.at[pl.ds(0, EPS)], bj[p], msem).wait()

    def issue(g, p, rb):
        pltpu.async_copy(table_sp.at[ix[p]], rows[rb], gsem)

    def compute(p, rb):
        def center(ci, carry):
            wv = []
            for h in range(M // 16):
                wiv = bi[p][pl.ds(ci * M + h * 16, 16)]
                wjv = bj[p][pl.ds(ci * M + h * 16, 16)]
                wv.append(wiv * wjv)
            g_acc = [jnp.zeros((16,), jnp.float32) for _ in range(D // 16)]
            for m in range(M):
                w = wv[m // 16][m % 16]
                for k in range(D // 16):
                    g_acc[k] = g_acc[k] + w * rows[rb][ci * M + m,
                                                       pl.ds(k * 16, 16)]
            for k in range(D // 16):
                go[rb][pl.ds(ci * D + k * 16, 16)] = g_acc[k]
            return carry

        lax.fori_loop(0, SPC, center, 0)

    for p in range(MD):
        meta_issue(p, p)
    plsc.subcore_barrier()
    meta_wait(0)
    issue(0, 0, 0)
    meta_wait(1)
    issue(1, 1, 1)

    def outer(t, carry):
        g0 = t * MD
        for b in range(MD):
            g = g0 + b
            rb = b % NB
            pltpu.make_async_copy(
                table_sp.at[ix[0]], rows[rb], gsem).wait()

            @pl.when(g >= NB)
            def _wait_out():
                pltpu.make_async_copy(
                    go[rb], g_hbm.at[pl.ds(0, SPC * D)], osem).wait()

            compute(b, rb)
            pltpu.async_copy(
                go[rb], g_hbm.at[pl.ds((cbase + g * SPC) * D, SPC * D)], osem)

            @pl.when(g + MD < NSTEP)
            def _issue_meta():
                meta_issue(g + MD, b)

            @pl.when(g + NB < NSTEP)
            def _issue_gather():
                meta_wait((b + NB) % MD)
                issue(g + NB, (b + NB) % MD, rb)
        return carry

    lax.fori_loop(0, NSTEP // MD, outer, 0)

    for rb in range(NB):
        pltpu.make_async_copy(go[rb], g_hbm.at[pl.ds(0, SPC * D)], osem).wait()


@jax.jit
def _sc_gather(atom_in_fea, idx_flat, bwi_flat, bwj_flat):
    mesh = plsc.VectorSubcoreMesh(core_axis_name="c", subcore_axis_name="s")
    f = pl.kernel(
        _sc_gather_kernel,
        out_type=jax.ShapeDtypeStruct((NPAD * D,), jnp.float32),
        mesh=mesh,
        scratch_types=(
            [pltpu.VMEM_SHARED((N, D), jnp.float32)]
            + [pltpu.VMEM((EPS,), jnp.int32)] * MD
            + [pltpu.VMEM((EPS,), jnp.float32)] * MD
            + [pltpu.VMEM((EPS,), jnp.float32)] * MD
            + [pltpu.VMEM((EPS, D), jnp.float32)] * NB
            + [pltpu.VMEM((SPC * D,), jnp.float32)] * NB
            + [pltpu.SemaphoreType.DMA] * 3
        ),
    )
    return f(atom_in_fea, idx_flat, bwi_flat, bwj_flat)


FB = 400


def _tc_f_kernel(bwi_ref, bwj_ref, nbr_ref, f_ref):
    w = bwi_ref[...] * bwj_ref[...]
    nb = nbr_ref[...]
    acc = nb[:, 0, :] * w[:, 0:1]
    for m in range(1, M):
        acc = acc + nb[:, m, :] * w[:, m:m + 1]
    f_ref[...] = acc


@jax.jit
def _tc_f(bwi, bwj, nbr_fea):
    return pl.pallas_call(
        _tc_f_kernel,
        grid=(N // FB,),
        in_specs=[
            pl.BlockSpec((FB, M), lambda i: (i, 0)),
            pl.BlockSpec((FB, M), lambda i: (i, 0)),
            pl.BlockSpec((FB, M, K), lambda i: (i, 0, 0)),
        ],
        out_specs=pl.BlockSpec((FB, K), lambda i: (i, 0)),
        out_shape=jax.ShapeDtypeStruct((N, K), jnp.float32),
    )(bwi, bwj, nbr_fea)


def _tc_tail_kernel(atom_ref, bwi_ref, bwj_ref, g_ref, f_ref, wc_ref, wn_ref,
                    wf_ref, b_ref, g1_ref, b1_ref, g2_ref, b2_ref, out_ref):
    atom = atom_ref[...]
    s = jnp.sum(bwi_ref[...] * bwj_ref[...], axis=1, keepdims=True)
    z = jnp.dot(atom * s, wc_ref[...], preferred_element_type=jnp.float32)
    z = z + jnp.dot(g_ref[...][:N], wn_ref[...],
                    preferred_element_type=jnp.float32)
    z = z + jnp.dot(f_ref[...], wf_ref[...],
                    preferred_element_type=jnp.float32)
    z = z + b_ref[...]

    mean1 = jnp.mean(z, axis=0, keepdims=True)
    zc = z - mean1
    var1 = jnp.mean(zc * zc, axis=0, keepdims=True)
    zn = zc * lax.rsqrt(var1 + 1e-5) * g1_ref[...] + b1_ref[...]

    filt = zn[:, :D]
    core = zn[:, D:]
    a = (1.0 / (1.0 + jnp.exp(-filt))) * (
        jnp.maximum(core, 0.0) + jnp.log1p(jnp.exp(-jnp.abs(core))))

    mean2 = jnp.mean(a, axis=0, keepdims=True)
    ac = a - mean2
    var2 = jnp.mean(ac * ac, axis=0, keepdims=True)
    an = ac * lax.rsqrt(var2 + 1e-5) * g2_ref[...] + b2_ref[...]
    out_ref[...] = jnp.maximum(an, 0.0) + jnp.log1p(jnp.exp(-jnp.abs(an)))


@jax.jit
def _tc_tail(atom_in_fea, bwi, bwj, G, F, WcT, WnT, WfT, b, g1, b1, g2, b2):
    return pl.pallas_call(
        _tc_tail_kernel,
        out_shape=jax.ShapeDtypeStruct((N, D), jnp.float32),
    )(atom_in_fea, bwi, bwj, G, F, WcT, WnT, WfT, b, g1, b1, g2, b2)


def kernel(atom_in_fea, nbr_fea, nbr_fea_idx, bond_weights_i, bond_weights_j,
           W_full, b_full, bn1_gamma, bn1_beta, bn2_gamma, bn2_beta):
    pad = NPAD - N
    idx_flat = jnp.pad(nbr_fea_idx.reshape(-1), (0, pad * M))
    bwi_flat = jnp.pad(bond_weights_i.reshape(-1), (0, pad * M))
    bwj_flat = jnp.pad(bond_weights_j.reshape(-1), (0, pad * M))

    G = _sc_gather(atom_in_fea, idx_flat, bwi_flat, bwj_flat)
    G = G.reshape(NPAD, D)
    F = _tc_f(bond_weights_i, bond_weights_j, nbr_fea)

    WT = W_full.T
    WcT = WT[:D]
    WnT = WT[D:2 * D]
    WfT = WT[2 * D:]
    return _tc_tail(atom_in_fea, bond_weights_i, bond_weights_j, G, F,
                    WcT, WnT, WfT, b_full[None, :],
                    bn1_gamma[None, :], bn1_beta[None, :],
                    bn2_gamma[None, :], bn2_beta[None, :])


# speedup vs baseline: 2.1599x; 2.1599x over previous
import jax
import jax.numpy as jnp
from jax import lax
from jax.experimental import pallas as pl
from jax.experimental.pallas import tpu as pltpu
from jax.experimental.pallas import tpu_sc as plsc

N = 10000
M = 32
D = 128
K = 16
NW = 32
C = 320
NPAD = NW * C
SPC = 4
EPS = SPC * M
NSTEP = C // SPC
NB = 2
MD = 4
NEDGE = N * M

import numpy as _np
_R_EXPAND = _np.zeros((M, M * K), _np.float32)
for _m in range(M):
    _R_EXPAND[_m, _m * K:(_m + 1) * K] = 1.0


def _sc_gather_kernel(atom_hbm, idx_hbm, bwi_hbm, bwj_hbm,
                      g_hbm,
                      table_sp,
                      ix0, ix1, ix2, ix3,
                      bi0, bi1, bi2, bi3,
                      bj0, bj1, bj2, bj3,
                      rows0, rows1, go0, go1,
                      gsem, osem, msem):
    ix = [ix0, ix1, ix2, ix3]
    bi = [bi0, bi1, bi2, bi3]
    bj = [bj0, bj1, bj2, bj3]
    rows = [rows0, rows1]
    go = [go0, go1]

    nc = 2
    wid = lax.axis_index("s") * nc + lax.axis_index("c")
    ebase = wid * (C * M)
    cbase = wid * C

    @pl.when(lax.axis_index("s") == 0)
    def _stage_table():
        pltpu.sync_copy(atom_hbm, table_sp)

    def meta_issue(g, p):
        e0 = ebase + g * EPS
        pltpu.async_copy(idx_hbm.at[pl.ds(e0, EPS)], ix[p], msem)
        pltpu.async_copy(bwi_hbm.at[pl.ds(e0, EPS)], bi[p], msem)
        pltpu.async_copy(bwj_hbm.at[pl.ds(e0, EPS)], bj[p], msem)

    def meta_wait(p):
        pltpu.make_async_copy(idx_hbm.at[pl.ds(0, EPS)], ix[p], msem).wait()
        pltpu.make_async_copy(bwi_hbm.at[pl.ds(0, EPS)], bi[p], msem).wait()
        pltpu.make_async_copy(bwj_hbm.at[pl.ds(0, EPS)], bj[p], msem).wait()

    def issue(g, p, rb):
        pltpu.async_copy(table_sp.at[ix[p]], rows[rb], gsem)

    def compute(p, rb):
        def center(ci, carry):
            wv = []
            for h in range(M // 16):
                wiv = bi[p][pl.ds(ci * M + h * 16, 16)]
                wjv = bj[p][pl.ds(ci * M + h * 16, 16)]
                wv.append(wiv * wjv)
            g_acc = [jnp.zeros((16,), jnp.float32) for _ in range(D // 16)]
            for m in range(M):
                w = wv[m // 16][m % 16]
                for k in range(D // 16):
                    g_acc[k] = g_acc[k] + w * rows[rb][ci * M + m,
                                                       pl.ds(k * 16, 16)]
            for k in range(D // 16):
                go[rb][pl.ds(ci * D + k * 16, 16)] = g_acc[k]
            return carry

        lax.fori_loop(0, SPC, center, 0)

    for p in range(MD):
        meta_issue(p, p)
    plsc.subcore_barrier()
    meta_wait(0)
    issue(0, 0, 0)
    meta_wait(1)
    issue(1, 1, 1)

    def outer(t, carry):
        g0 = t * MD
        for b in range(MD):
            g = g0 + b
            rb = b % NB
            pltpu.make_async_copy(
                table_sp.at[ix[0]], rows[rb], gsem).wait()

            @pl.when(g >= NB)
            def _wait_out():
                pltpu.make_async_copy(
                    go[rb], g_hbm.at[pl.ds(0, SPC * D)], osem).wait()

            compute(b, rb)
            pltpu.async_copy(
                go[rb], g_hbm.at[pl.ds((cbase + g * SPC) * D, SPC * D)], osem)

            @pl.when(g + MD < NSTEP)
            def _issue_meta():
                meta_issue(g + MD, b)

            @pl.when(g + NB < NSTEP)
            def _issue_gather():
                meta_wait((b + NB) % MD)
                issue(g + NB, (b + NB) % MD, rb)
        return carry

    lax.fori_loop(0, NSTEP // MD, outer, 0)

    for rb in range(NB):
        pltpu.make_async_copy(go[rb], g_hbm.at[pl.ds(0, SPC * D)], osem).wait()


@jax.jit
def _sc_gather(atom_in_fea, idx_flat, bwi_flat, bwj_flat):
    mesh = plsc.VectorSubcoreMesh(core_axis_name="c", subcore_axis_name="s")
    f = pl.kernel(
        _sc_gather_kernel,
        out_type=jax.ShapeDtypeStruct((NPAD * D,), jnp.float32),
        mesh=mesh,
        scratch_types=(
            [pltpu.VMEM_SHARED((N, D), jnp.float32)]
            + [pltpu.VMEM((EPS,), jnp.int32)] * MD
            + [pltpu.VMEM((EPS,), jnp.float32)] * MD
            + [pltpu.VMEM((EPS,), jnp.float32)] * MD
            + [pltpu.VMEM((EPS, D), jnp.float32)] * NB
            + [pltpu.VMEM((SPC * D,), jnp.float32)] * NB
            + [pltpu.SemaphoreType.DMA] * 3
        ),
    )
    return f(atom_in_fea, idx_flat, bwi_flat, bwj_flat)


ZB = 2000


def _tc_z_kernel(atom_ref, bwi_ref, bwj_ref, g_ref, nbr_ref, r_ref,
                 wc_ref, wn_ref, wfb_ref, b_ref, z_ref):
    w = bwi_ref[...] * bwj_ref[...]
    s = jnp.sum(w, axis=1, keepdims=True)
    z = jnp.dot(atom_ref[...] * s, wc_ref[...],
                preferred_element_type=jnp.float32)
    z = z + jnp.dot(g_ref[...], wn_ref[...],
                    preferred_element_type=jnp.float32)
    wexp = jnp.dot(w, r_ref[...], preferred_element_type=jnp.float32)
    z = z + jnp.dot(nbr_ref[...] * wexp, wfb_ref[...],
                    preferred_element_type=jnp.float32)
    z_ref[...] = z + b_ref[...]


def _tc_bn_kernel(z_ref, g1_ref, b1_ref, g2_ref, b2_ref, out_ref):
    z = z_ref[...]
    mean1 = jnp.mean(z, axis=0, keepdims=True)
    zc = z - mean1
    var1 = jnp.mean(zc * zc, axis=0, keepdims=True)
    zn = zc * lax.rsqrt(var1 + 1e-5) * g1_ref[...] + b1_ref[...]

    filt = zn[:, :D]
    core = zn[:, D:]
    a = (1.0 / (1.0 + jnp.exp(-filt))) * (
        jnp.maximum(core, 0.0) + jnp.log1p(jnp.exp(-jnp.abs(core))))

    mean2 = jnp.mean(a, axis=0, keepdims=True)
    ac = a - mean2
    var2 = jnp.mean(ac * ac, axis=0, keepdims=True)
    an = ac * lax.rsqrt(var2 + 1e-5) * g2_ref[...] + b2_ref[...]
    out_ref[...] = jnp.maximum(an, 0.0) + jnp.log1p(jnp.exp(-jnp.abs(an)))


@jax.jit
def _tc_tail(atom_in_fea, bwi, bwj, G, nbr2, R, WcT, WnT, Wfb, b,
             g1, b1, g2, b2):
    z = pl.pallas_call(
        _tc_z_kernel,
        grid=(N // ZB,),
        in_specs=[
            pl.BlockSpec((ZB, D), lambda i: (i, 0)),
            pl.BlockSpec((ZB, M), lambda i: (i, 0)),
            pl.BlockSpec((ZB, M), lambda i: (i, 0)),
            pl.BlockSpec((ZB, D), lambda i: (i, 0)),
            pl.BlockSpec((ZB, M * K), lambda i: (i, 0)),
            pl.BlockSpec((M, M * K), lambda i: (0, 0)),
            pl.BlockSpec((D, 2 * D), lambda i: (0, 0)),
            pl.BlockSpec((D, 2 * D), lambda i: (0, 0)),
            pl.BlockSpec((M * K, 2 * D), lambda i: (0, 0)),
            pl.BlockSpec((1, 2 * D), lambda i: (0, 0)),
        ],
        out_specs=pl.BlockSpec((ZB, 2 * D), lambda i: (i, 0)),
        out_shape=jax.ShapeDtypeStruct((N, 2 * D), jnp.float32),
    )(atom_in_fea, bwi, bwj, G, nbr2, R, WcT, WnT, Wfb, b)
    return pl.pallas_call(
        _tc_bn_kernel,
        out_shape=jax.ShapeDtypeStruct((N, D), jnp.float32),
    )(z, g1, b1, g2, b2)


def kernel(atom_in_fea, nbr_fea, nbr_fea_idx, bond_weights_i, bond_weights_j,
           W_full, b_full, bn1_gamma, bn1_beta, bn2_gamma, bn2_beta):
    pad = NPAD - N
    idx_flat = jnp.pad(nbr_fea_idx.reshape(-1), (0, pad * M))
    bwi_flat = jnp.pad(bond_weights_i.reshape(-1), (0, pad * M))
    bwj_flat = jnp.pad(bond_weights_j.reshape(-1), (0, pad * M))

    G = _sc_gather(atom_in_fea, idx_flat, bwi_flat, bwj_flat)
    G = G.reshape(NPAD, D)
    nbr2 = nbr_fea.reshape(N, M * K)

    WT = W_full.T
    WcT = WT[:D]
    WnT = WT[D:2 * D]
    WfT = WT[2 * D:]
    Wfb = jnp.tile(WfT, (M, 1))
    R = jnp.asarray(_R_EXPAND)
    return _tc_tail(atom_in_fea, bond_weights_i, bond_weights_j, G[:N], nbr2,
                    R, WcT, WnT, Wfb, b_full[None, :],
                    bn1_gamma[None, :], bn1_beta[None, :],
                    bn2_gamma[None, :], bn2_beta[None, :])


# speedup vs baseline: 2.2225x; 1.0290x over previous
import jax
import jax.numpy as jnp
from jax import lax
from jax.experimental import pallas as pl
from jax.experimental.pallas import tpu as pltpu
from jax.experimental.pallas import tpu_sc as plsc

N = 10000
M = 32
D = 128
K = 16
NW = 32
C = 320
NPAD = NW * C
SPC = 4
EPS = SPC * M
NSTEP = C // SPC
NB = 2
MD = 4
NEDGE = N * M

import numpy as _np
_R_EXPAND = _np.zeros((M, M * K), _np.float32)
for _m in range(M):
    _R_EXPAND[_m, _m * K:(_m + 1) * K] = 1.0


def _sc_gather_kernel(atom_hbm, idx_hbm, bwi_hbm, bwj_hbm,
                      g_hbm,
                      table_sp,
                      ix0, ix1, ix2, ix3,
                      bi0, bi1, bi2, bi3,
                      bj0, bj1, bj2, bj3,
                      rows0, rows1, go0, go1,
                      gsem, osem, msem):
    ix = [ix0, ix1, ix2, ix3]
    bi = [bi0, bi1, bi2, bi3]
    bj = [bj0, bj1, bj2, bj3]
    rows = [rows0, rows1]
    go = [go0, go1]

    nc = 2
    wid = lax.axis_index("s") * nc + lax.axis_index("c")
    ebase = wid * (C * M)
    cbase = wid * C

    @pl.when(lax.axis_index("s") == 0)
    def _stage_table():
        pltpu.sync_copy(atom_hbm, table_sp)

    def meta_issue(g, p):
        e0 = ebase + g * EPS
        pltpu.async_copy(idx_hbm.at[pl.ds(e0, EPS)], ix[p], msem)
        pltpu.async_copy(bwi_hbm.at[pl.ds(e0, EPS)], bi[p], msem)
        pltpu.async_copy(bwj_hbm.at[pl.ds(e0, EPS)], bj[p], msem)

    def meta_wait(p):
        pltpu.make_async_copy(idx_hbm.at[pl.ds(0, EPS)], ix[p], msem).wait()
        pltpu.make_async_copy(bwi_hbm.at[pl.ds(0, EPS)], bi[p], msem).wait()
        pltpu.make_async_copy(bwj_hbm.at[pl.ds(0, EPS)], bj[p], msem).wait()

    def issue(g, p, rb):
        pltpu.async_copy(table_sp.at[ix[p]], rows[rb], gsem)

    def compute(p, rb):
        def center(ci, carry):
            wv = []
            for h in range(M // 16):
                wiv = bi[p][pl.ds(ci * M + h * 16, 16)]
                wjv = bj[p][pl.ds(ci * M + h * 16, 16)]
                wv.append(wiv * wjv)
            g_acc = [jnp.zeros((16,), jnp.float32) for _ in range(D // 16)]
            for m in range(M):
                w = wv[m // 16][m % 16]
                for k in range(D // 16):
                    g_acc[k] = g_acc[k] + w * rows[rb][ci * M + m,
                                                       pl.ds(k * 16, 16)]
            for k in range(D // 16):
                go[rb][pl.ds(ci * D + k * 16, 16)] = g_acc[k]
            return carry

        lax.fori_loop(0, SPC, center, 0)

    for p in range(MD):
        meta_issue(p, p)
    plsc.subcore_barrier()
    meta_wait(0)
    issue(0, 0, 0)
    meta_wait(1)
    issue(1, 1, 1)

    def outer(t, carry):
        g0 = t * MD
        for b in range(MD):
            g = g0 + b
            rb = b % NB
            pltpu.make_async_copy(
                table_sp.at[ix[0]], rows[rb], gsem).wait()

            @pl.when(g >= NB)
            def _wait_out():
                pltpu.make_async_copy(
                    go[rb], g_hbm.at[pl.ds(0, SPC * D)], osem).wait()

            compute(b, rb)
            pltpu.async_copy(
                go[rb], g_hbm.at[pl.ds((cbase + g * SPC) * D, SPC * D)], osem)

            @pl.when(g + MD < NSTEP)
            def _issue_meta():
                meta_issue(g + MD, b)

            @pl.when(g + NB < NSTEP)
            def _issue_gather():
                meta_wait((b + NB) % MD)
                issue(g + NB, (b + NB) % MD, rb)
        return carry

    lax.fori_loop(0, NSTEP // MD, outer, 0)

    for rb in range(NB):
        pltpu.make_async_copy(go[rb], g_hbm.at[pl.ds(0, SPC * D)], osem).wait()


@jax.jit
def _sc_gather(atom_in_fea, idx_flat, bwi_flat, bwj_flat):
    mesh = plsc.VectorSubcoreMesh(core_axis_name="c", subcore_axis_name="s")
    f = pl.kernel(
        _sc_gather_kernel,
        out_type=jax.ShapeDtypeStruct((NPAD * D,), jnp.float32),
        mesh=mesh,
        scratch_types=(
            [pltpu.VMEM_SHARED((N, D), jnp.float32)]
            + [pltpu.VMEM((EPS,), jnp.int32)] * MD
            + [pltpu.VMEM((EPS,), jnp.float32)] * MD
            + [pltpu.VMEM((EPS,), jnp.float32)] * MD
            + [pltpu.VMEM((EPS, D), jnp.float32)] * NB
            + [pltpu.VMEM((SPC * D,), jnp.float32)] * NB
            + [pltpu.SemaphoreType.DMA] * 3
        ),
    )
    return f(atom_in_fea, idx_flat, bwi_flat, bwj_flat)


ZB = 2000


def _tc_z_kernel(atom_ref, bwi_ref, bwj_ref, g_ref, nbr_ref, r_ref,
                 wc_ref, wn_ref, wfb_ref, b_ref, z_ref):
    w = bwi_ref[...] * bwj_ref[...]
    s = jnp.sum(w, axis=1, keepdims=True)
    z = jnp.dot(atom_ref[...] * s, wc_ref[...],
                preferred_element_type=jnp.float32)
    z = z + jnp.dot(g_ref[...], wn_ref[...],
                    preferred_element_type=jnp.float32)
    wexp = jnp.dot(w, r_ref[...], preferred_element_type=jnp.float32)
    z = z + jnp.dot(nbr_ref[...] * wexp, wfb_ref[...],
                    preferred_element_type=jnp.float32)
    z_ref[...] = z + b_ref[...]


def _tc_bn_kernel(z_ref, g1_ref, b1_ref, g2_ref, b2_ref, out_ref):
    z = z_ref[...]
    mean1 = jnp.mean(z, axis=0, keepdims=True)
    zc = z - mean1
    var1 = jnp.mean(zc * zc, axis=0, keepdims=True)
    zn = zc * lax.rsqrt(var1 + 1e-5) * g1_ref[...] + b1_ref[...]

    filt = zn[:, :D]
    core = zn[:, D:]
    a = (1.0 / (1.0 + jnp.exp(-filt))) * (
        jnp.maximum(core, 0.0) + jnp.log1p(jnp.exp(-jnp.abs(core))))

    mean2 = jnp.mean(a, axis=0, keepdims=True)
    ac = a - mean2
    var2 = jnp.mean(ac * ac, axis=0, keepdims=True)
    an = ac * lax.rsqrt(var2 + 1e-5) * g2_ref[...] + b2_ref[...]
    out_ref[...] = jnp.maximum(an, 0.0) + jnp.log1p(jnp.exp(-jnp.abs(an)))


@jax.jit
def _tc_tail(atom_in_fea, bwi, bwj, G, nbr2, R, WcT, WnT, Wfb, b,
             g1, b1, g2, b2):
    z = pl.pallas_call(
        _tc_z_kernel,
        grid=(N // ZB,),
        in_specs=[
            pl.BlockSpec((ZB, D), lambda i: (i, 0)),
            pl.BlockSpec((ZB, M), lambda i: (i, 0)),
            pl.BlockSpec((ZB, M), lambda i: (i, 0)),
            pl.BlockSpec((ZB, D), lambda i: (i, 0)),
            pl.BlockSpec((ZB, M * K), lambda i: (i, 0)),
            pl.BlockSpec((M, M * K), lambda i: (0, 0)),
            pl.BlockSpec((D, 2 * D), lambda i: (0, 0)),
            pl.BlockSpec((D, 2 * D), lambda i: (0, 0)),
            pl.BlockSpec((M * K, 2 * D), lambda i: (0, 0)),
            pl.BlockSpec((1, 2 * D), lambda i: (0, 0)),
        ],
        out_specs=pl.BlockSpec((ZB, 2 * D), lambda i: (i, 0)),
        out_shape=jax.ShapeDtypeStruct((N, 2 * D), jnp.float32),
    )(atom_in_fea, bwi, bwj, G, nbr2, R, WcT, WnT, Wfb, b)
    return pl.pallas_call(
        _tc_bn_kernel,
        out_shape=jax.ShapeDtypeStruct((N, D), jnp.float32),
    )(z, g1, b1, g2, b2)


def kernel(atom_in_fea, nbr_fea, nbr_fea_idx, bond_weights_i, bond_weights_j,
           W_full, b_full, bn1_gamma, bn1_beta, bn2_gamma, bn2_beta):
    pad = NPAD - N
    idx_flat = jnp.pad(nbr_fea_idx.reshape(-1), (0, pad * M))
    bwi_flat = jnp.pad(bond_weights_i.reshape(-1), (0, pad * M))
    bwj_flat = jnp.pad(bond_weights_j.reshape(-1), (0, pad * M))

    G = _sc_gather(atom_in_fea, idx_flat, bwi_flat, bwj_flat)
    G = G.reshape(NPAD, D)
    nbr2 = nbr_fea.reshape(N, M * K)

    WT = W_full.T
    WcT = WT[:D]
    WnT = WT[D:2 * D]
    WfT = WT[2 * D:]
    Wfb = jnp.tile(WfT, (M, 1))
    R = jnp.asarray(_R_EXPAND)
    return _tc_tail(atom_in_fea, bond_weights_i, bond_weights_j, G, nbr2,
                    R, WcT, WnT, Wfb, b_full[None, :],
                    bn1_gamma[None, :], bn1_beta[None, :],
                    bn2_gamma[None, :], bn2_beta[None, :])


# speedup vs baseline: 2.2244x; 1.0009x over previous
import jax
import jax.numpy as jnp
import numpy as _np
from jax import lax
from jax.experimental import pallas as pl
from jax.experimental.pallas import tpu as pltpu
from jax.experimental.pallas import tpu_sc as plsc

N = 10000
M = 32
D = 128
K = 16
NW = 32
C = 320
NPAD = NW * C
SPC = 4
EPS = SPC * M
NSTEP = C // SPC
NB = 2
MD = 4
_R_EXPAND = _np.zeros((M, M * K), _np.float32)
for _m in range(M):
    _R_EXPAND[_m, _m * K:(_m + 1) * K] = 1.0


def _sc_gather_kernel(atom_hbm, idx_hbm, bwi_hbm, bwj_hbm,
                      g_hbm,
                      table_sp,
                      ix0, ix1, ix2, ix3,
                      bi0, bi1, bi2, bi3,
                      bj0, bj1, bj2, bj3,
                      rows0, rows1, go0, go1,
                      gsem, osem, msem):
    ix = [ix0, ix1, ix2, ix3]
    bi = [bi0, bi1, bi2, bi3]
    bj = [bj0, bj1, bj2, bj3]
    rows = [rows0, rows1]
    go = [go0, go1]

    nc = 2
    wid = lax.axis_index("s") * nc + lax.axis_index("c")
    ebase = wid * (C * M)
    cbase = wid * C

    @pl.when(lax.axis_index("s") == 0)
    def _stage_table():
        pltpu.sync_copy(atom_hbm, table_sp)

    def meta_issue(g, p):
        e0 = ebase + g * EPS
        pltpu.async_copy(idx_hbm.at[pl.ds(e0, EPS)], ix[p], msem)
        pltpu.async_copy(bwi_hbm.at[pl.ds(e0, EPS)], bi[p], msem)
        pltpu.async_copy(bwj_hbm.at[pl.ds(e0, EPS)], bj[p], msem)

    def meta_wait(p):
        pltpu.make_async_copy(idx_hbm.at[pl.ds(0, EPS)], ix[p], msem).wait()
        pltpu.make_async_copy(bwi_hbm.at[pl.ds(0, EPS)], bi[p], msem).wait()
        pltpu.make_async_copy(bwj_hbm.at[pl.ds(0, EPS)], bj[p], msem).wait()

    def issue(g, p, rb):
        pltpu.async_copy(table_sp.at[ix[p]], rows[rb], gsem)

    def compute(p, rb):
        def center(ci, carry):
            wv = []
            for h in range(M // 16):
                wiv = bi[p][pl.ds(ci * M + h * 16, 16)]
                wjv = bj[p][pl.ds(ci * M + h * 16, 16)]
                wv.append(wiv * wjv)
            g_acc = [jnp.zeros((16,), jnp.float32) for _ in range(D // 16)]
            for m in range(M):
                w = wv[m // 16][m % 16]
                for k in range(D // 16):
                    g_acc[k] = g_acc[k] + w * rows[rb][ci * M + m,
                                                       pl.ds(k * 16, 16)]
            for k in range(D // 16):
                go[rb][pl.ds(ci * D + k * 16, 16)] = g_acc[k]
            return carry

        lax.fori_loop(0, SPC, center, 0)

    for p in range(MD):
        meta_issue(p, p)
    plsc.subcore_barrier()
    meta_wait(0)
    issue(0, 0, 0)
    meta_wait(1)
    issue(1, 1, 1)

    def outer(t, carry):
        g0 = t * MD
        for b in range(MD):
            g = g0 + b
            rb = b % NB
            pltpu.make_async_copy(
                table_sp.at[ix[0]], rows[rb], gsem).wait()

            @pl.when(g >= NB)
            def _wait_out():
                pltpu.make_async_copy(
                    go[rb], g_hbm.at[pl.ds(0, SPC * D)], osem).wait()

            compute(b, rb)
            pltpu.async_copy(
                go[rb], g_hbm.at[pl.ds((cbase + g * SPC) * D, SPC * D)], osem)

            @pl.when(g + MD < NSTEP)
            def _issue_meta():
                meta_issue(g + MD, b)

            @pl.when(g + NB < NSTEP)
            def _issue_gather():
                meta_wait((b + NB) % MD)
                issue(g + NB, (b + NB) % MD, rb)
        return carry

    lax.fori_loop(0, NSTEP // MD, outer, 0)

    for rb in range(NB):
        pltpu.make_async_copy(go[rb], g_hbm.at[pl.ds(0, SPC * D)], osem).wait()


@jax.jit
def _sc_gather(atom_in_fea, idx_flat, bwi_flat, bwj_flat):
    mesh = plsc.VectorSubcoreMesh(core_axis_name="c", subcore_axis_name="s")
    f = pl.kernel(
        _sc_gather_kernel,
        out_type=jax.ShapeDtypeStruct((NPAD * D,), jnp.float32),
        mesh=mesh,
        scratch_types=(
            [pltpu.VMEM_SHARED((N, D), jnp.float32)]
            + [pltpu.VMEM((EPS,), jnp.int32)] * MD
            + [pltpu.VMEM((EPS,), jnp.float32)] * MD
            + [pltpu.VMEM((EPS,), jnp.float32)] * MD
            + [pltpu.VMEM((EPS, D), jnp.float32)] * NB
            + [pltpu.VMEM((SPC * D,), jnp.float32)] * NB
            + [pltpu.SemaphoreType.DMA] * 3
        ),
    )
    return f(atom_in_fea, idx_flat, bwi_flat, bwj_flat)


ZB = 2000


def _tc_z_kernel(atom_ref, bwi_ref, bwj_ref, g_ref, nbr_ref, r_ref,
                 wc_ref, wn_ref, wfb_ref, b_ref, z_ref):
    w = bwi_ref[...] * bwj_ref[...]
    s = jnp.sum(w, axis=1, keepdims=True)
    z = jnp.dot(atom_ref[...] * s, wc_ref[...],
                preferred_element_type=jnp.float32)
    z = z + jnp.dot(g_ref[...], wn_ref[...],
                    preferred_element_type=jnp.float32)
    wexp = jnp.dot(w, r_ref[...], preferred_element_type=jnp.float32)
    z = z + jnp.dot(nbr_ref[...] * wexp, wfb_ref[...],
                    preferred_element_type=jnp.float32)
    z_ref[...] = z + b_ref[...]


def _tc_bn_kernel(z_ref, g1_ref, b1_ref, g2_ref, b2_ref, out_ref):
    z = z_ref[...]
    mean1 = jnp.mean(z, axis=0, keepdims=True)
    zc = z - mean1
    var1 = jnp.mean(zc * zc, axis=0, keepdims=True)
    zn = zc * lax.rsqrt(var1 + 1e-5) * g1_ref[...] + b1_ref[...]

    filt = zn[:, :D]
    core = zn[:, D:]
    a = (1.0 / (1.0 + jnp.exp(-filt))) * (
        jnp.maximum(core, 0.0) + jnp.log1p(jnp.exp(-jnp.abs(core))))

    mean2 = jnp.mean(a, axis=0, keepdims=True)
    ac = a - mean2
    var2 = jnp.mean(ac * ac, axis=0, keepdims=True)
    an = ac * lax.rsqrt(var2 + 1e-5) * g2_ref[...] + b2_ref[...]
    out_ref[...] = jnp.maximum(an, 0.0) + jnp.log1p(jnp.exp(-jnp.abs(an)))


@jax.jit
def _tc_tail(atom_in_fea, bwi, bwj, G, nbr2, R, WcT, WnT, Wfb, b,
             g1, b1, g2, b2):
    z = pl.pallas_call(
        _tc_z_kernel,
        grid=(N // ZB,),
        in_specs=[
            pl.BlockSpec((ZB, D), lambda i: (i, 0)),
            pl.BlockSpec((ZB, M), lambda i: (i, 0)),
            pl.BlockSpec((ZB, M), lambda i: (i, 0)),
            pl.BlockSpec((ZB, D), lambda i: (i, 0)),
            pl.BlockSpec((ZB, M * K), lambda i: (i, 0)),
            pl.BlockSpec((M, M * K), lambda i: (0, 0)),
            pl.BlockSpec((D, 2 * D), lambda i: (0, 0)),
            pl.BlockSpec((D, 2 * D), lambda i: (0, 0)),
            pl.BlockSpec((M * K, 2 * D), lambda i: (0, 0)),
            pl.BlockSpec((1, 2 * D), lambda i: (0, 0)),
        ],
        out_specs=pl.BlockSpec((ZB, 2 * D), lambda i: (i, 0)),
        out_shape=jax.ShapeDtypeStruct((N, 2 * D), jnp.float32),
    )(atom_in_fea, bwi, bwj, G, nbr2, R, WcT, WnT, Wfb, b)
    return pl.pallas_call(
        _tc_bn_kernel,
        out_shape=jax.ShapeDtypeStruct((N, D), jnp.float32),
    )(z, g1, b1, g2, b2)


def kernel(atom_in_fea, nbr_fea, nbr_fea_idx, bond_weights_i, bond_weights_j,
           W_full, b_full, bn1_gamma, bn1_beta, bn2_gamma, bn2_beta):
    pad = NPAD - N
    idx_flat = jnp.pad(nbr_fea_idx.reshape(-1), (0, pad * M))
    bwi_flat = jnp.pad(bond_weights_i.reshape(-1), (0, pad * M))
    bwj_flat = jnp.pad(bond_weights_j.reshape(-1), (0, pad * M))

    G = _sc_gather(atom_in_fea, idx_flat, bwi_flat, bwj_flat)
    G = G.reshape(NPAD, D)
    nbr2 = nbr_fea.reshape(N, M * K)

    WT = W_full.T
    WcT = WT[:D]
    WnT = WT[D:2 * D]
    WfT = WT[2 * D:]
    Wfb = jnp.tile(WfT, (M, 1))
    R = jnp.asarray(_R_EXPAND)
    return _tc_tail(atom_in_fea, bond_weights_i, bond_weights_j, G, nbr2,
                    R, WcT, WnT, Wfb, b_full[None, :],
                    bn1_gamma[None, :], bn1_beta[None, :],
                    bn2_gamma[None, :], bn2_beta[None, :])
